# baseline (speedup 1.0000x reference)
import jax
import jax.numpy as jnp
from jax import lax
from jax.experimental import pallas as pl
from jax.experimental.pallas import tpu as pltpu

_NUM, _P, _C = 32, 8732, 21
_BN = 8
_PBL = 384
_GB, _GP = _NUM // _BN, 23


def _body(prior_ref,
          lm1, lv1, lp1, lm2, lv2, lp2, lm3, lv3, lp3, lm4, lv4, lp4,
          cm1, cv1, cp1, cm2, cv2, cp2, cm3, cv3, cp3, cm4, cv4, cp4,
          out_ref):
    m1, m2, m3, m4 = lm1[...], lm2[...], lm3[...], lm4[...]
    w1, w2, w3, w4 = lp1[...], lp2[...], lp3[...], lp4[...]
    s1, s2, s3, s4 = lv1[...], lv2[...], lv3[...], lv4[...]
    nl = w1 * m1 + w2 * m2 + w3 * m3 + w4 * m4
    al = w1 * s1 + w2 * s2 + w3 * s3 + w4 * s4
    ep = (w1 * (m1 - nl) ** 2 + w2 * (m2 - nl) ** 2
          + w3 * (m3 - nl) ** 2 + w4 * (m4 - nl) ** 2)

    pr = prior_ref[...]
    prw = jnp.roll(pr, -2, axis=0)
    nlw = jnp.roll(nl, -2, axis=1)
    wh = prw[None] * jnp.exp(nlw * 0.2)
    x1y1 = pr[None] + nl * 0.1 * prw[None] - 0.5 * wh
    x2y2 = jnp.roll(x1y1 + wh, 2, axis=1)
    c_idx = lax.broadcasted_iota(jnp.int32, nl.shape, 1)
    dec = jnp.where(c_idx < 2, x1y1, x2y2)

    for c in range(4):
        out_ref[c] = dec[:, c, :]
        out_ref[4 + c] = al[:, c, :]
        out_ref[8 + c] = ep[:, c, :]

    a1, a2, a3, a4 = cm1[...], cm2[...], cm3[...], cm4[...]
    q1, q2, q3, q4 = cp1[...], cp2[...], cp3[...], cp4[...]
    t1, t2, t3, t4 = cv1[...], cv2[...], cv3[...], cv4[...]
    nc = q1 * a1 + q2 * a2 + q3 * a3 + q4 * a4
    ca = q1 * t1 + q2 * t2 + q3 * t3 + q4 * t4
    ce = (q1 * (a1 - nc) ** 2 + q2 * (a2 - nc) ** 2
          + q3 * (a3 - nc) ** 2 + q4 * (a4 - nc) ** 2)
    out_ref[12:33] = nc
    out_ref[33:54] = ca
    out_ref[54:75] = ce


def kernel(prior_data, loc_mu_1, loc_var_1, loc_pi_1, loc_mu_2, loc_var_2,
           loc_pi_2, loc_mu_3, loc_var_3, loc_pi_3, loc_mu_4, loc_var_4,
           loc_pi_4, conf_mu_1, conf_var_1, conf_pi_1, conf_mu_2, conf_var_2,
           conf_pi_2, conf_mu_3, conf_var_3, conf_pi_3, conf_mu_4, conf_var_4,
           conf_pi_4):
    locs = [loc_mu_1, loc_var_1, loc_pi_1, loc_mu_2, loc_var_2, loc_pi_2,
            loc_mu_3, loc_var_3, loc_pi_3, loc_mu_4, loc_var_4, loc_pi_4]
    confs = [conf_mu_1, conf_var_1, conf_pi_1, conf_mu_2, conf_var_2,
             conf_pi_2, conf_mu_3, conf_var_3, conf_pi_3, conf_mu_4,
             conf_var_4, conf_pi_4]
    locs = [jnp.transpose(x, (0, 2, 1)) for x in locs]
    confs = [jnp.transpose(x, (2, 0, 1)) for x in confs]
    prior_t = prior_data.T

    loc_spec = pl.BlockSpec((_BN, 4, _PBL), lambda b, j: (b, 0, j))
    conf_spec = pl.BlockSpec((_C, _BN, _PBL), lambda b, j: (0, b, j))
    prior_spec = pl.BlockSpec((4, _PBL), lambda b, j: (0, j))
    out_t = pl.pallas_call(
        _body,
        grid=(_GB, _GP),
        in_specs=[prior_spec] + [loc_spec] * 12 + [conf_spec] * 12,
        out_specs=pl.BlockSpec((75, _BN, _PBL), lambda b, j: (0, b, j)),
        out_shape=jax.ShapeDtypeStruct((75, _NUM, _P), jnp.float32),
        compiler_params=pltpu.CompilerParams(
            dimension_semantics=("parallel", "arbitrary")),
    )(prior_t, *locs, *confs)
    return jnp.transpose(out_t, (1, 2, 0))


# speedup vs baseline: 1.1309x; 1.1309x over previous
import jax
import jax.numpy as jnp
from jax import lax
from jax.experimental import pallas as pl
from jax.experimental.pallas import tpu as pltpu

_NUM, _P, _C = 32, 8732, 21
_BN = 8
_PBL = 1280
_GB, _GP = _NUM // _BN, 7


def _body(prior_ref,
          lm1, lv1, lp1, lm2, lv2, lp2, lm3, lv3, lp3, lm4, lv4, lp4,
          cm1, cv1, cp1, cm2, cv2, cp2, cm3, cv3, cp3, cm4, cv4, cp4,
          out_ref):
    m1, m2, m3, m4 = lm1[...], lm2[...], lm3[...], lm4[...]
    w1, w2, w3, w4 = lp1[...], lp2[...], lp3[...], lp4[...]
    s1, s2, s3, s4 = lv1[...], lv2[...], lv3[...], lv4[...]
    nl = w1 * m1 + w2 * m2 + w3 * m3 + w4 * m4
    al = w1 * s1 + w2 * s2 + w3 * s3 + w4 * s4
    ep = (w1 * (m1 - nl) ** 2 + w2 * (m2 - nl) ** 2
          + w3 * (m3 - nl) ** 2 + w4 * (m4 - nl) ** 2)

    pr = prior_ref[...]
    prw = jnp.roll(pr, -2, axis=0)
    nlw = jnp.roll(nl, -2, axis=1)
    wh = prw[None] * jnp.exp(nlw * 0.2)
    x1y1 = pr[None] + nl * 0.1 * prw[None] - 0.5 * wh
    x2y2 = jnp.roll(x1y1 + wh, 2, axis=1)
    c_idx = lax.broadcasted_iota(jnp.int32, nl.shape, 1)
    dec = jnp.where(c_idx < 2, x1y1, x2y2)

    for c in range(4):
        out_ref[c] = dec[:, c, :]
        out_ref[4 + c] = al[:, c, :]
        out_ref[8 + c] = ep[:, c, :]

    a1, a2, a3, a4 = cm1[...], cm2[...], cm3[...], cm4[...]
    q1, q2, q3, q4 = cp1[...], cp2[...], cp3[...], cp4[...]
    t1, t2, t3, t4 = cv1[...], cv2[...], cv3[...], cv4[...]
    nc = q1 * a1 + q2 * a2 + q3 * a3 + q4 * a4
    ca = q1 * t1 + q2 * t2 + q3 * t3 + q4 * t4
    ce = (q1 * (a1 - nc) ** 2 + q2 * (a2 - nc) ** 2
          + q3 * (a3 - nc) ** 2 + q4 * (a4 - nc) ** 2)
    out_ref[12:33] = nc
    out_ref[33:54] = ca
    out_ref[54:75] = ce


def kernel(prior_data, loc_mu_1, loc_var_1, loc_pi_1, loc_mu_2, loc_var_2,
           loc_pi_2, loc_mu_3, loc_var_3, loc_pi_3, loc_mu_4, loc_var_4,
           loc_pi_4, conf_mu_1, conf_var_1, conf_pi_1, conf_mu_2, conf_var_2,
           conf_pi_2, conf_mu_3, conf_var_3, conf_pi_3, conf_mu_4, conf_var_4,
           conf_pi_4):
    locs = [loc_mu_1, loc_var_1, loc_pi_1, loc_mu_2, loc_var_2, loc_pi_2,
            loc_mu_3, loc_var_3, loc_pi_3, loc_mu_4, loc_var_4, loc_pi_4]
    confs = [conf_mu_1, conf_var_1, conf_pi_1, conf_mu_2, conf_var_2,
             conf_pi_2, conf_mu_3, conf_var_3, conf_pi_3, conf_mu_4,
             conf_var_4, conf_pi_4]
    locs = [jnp.transpose(x, (0, 2, 1)) for x in locs]
    confs = [jnp.transpose(x, (2, 0, 1)) for x in confs]
    prior_t = prior_data.T

    loc_spec = pl.BlockSpec((_BN, 4, _PBL), lambda b, j: (b, 0, j))
    conf_spec = pl.BlockSpec((_C, _BN, _PBL), lambda b, j: (0, b, j))
    prior_spec = pl.BlockSpec((4, _PBL), lambda b, j: (0, j))
    out_t = pl.pallas_call(
        _body,
        grid=(_GB, _GP),
        in_specs=[prior_spec] + [loc_spec] * 12 + [conf_spec] * 12,
        out_specs=pl.BlockSpec((75, _BN, _PBL), lambda b, j: (0, b, j)),
        out_shape=jax.ShapeDtypeStruct((75, _NUM, _P), jnp.float32),
        compiler_params=pltpu.CompilerParams(
            dimension_semantics=("parallel", "arbitrary")),
    )(prior_t, *locs, *confs)
    return jnp.transpose(out_t, (1, 2, 0))


# speedup vs baseline: 1.1349x; 1.0036x over previous
import jax
import jax.numpy as jnp
from jax import lax
from jax.experimental import pallas as pl
from jax.experimental.pallas import tpu as pltpu

_NUM, _P, _C = 32, 8732, 21
_BN = 16
_PBL = 640
_GB, _GP = _NUM // _BN, 14


def _body(prior_ref,
          lm1, lv1, lp1, lm2, lv2, lp2, lm3, lv3, lp3, lm4, lv4, lp4,
          cm1, cv1, cp1, cm2, cv2, cp2, cm3, cv3, cp3, cm4, cv4, cp4,
          out_ref):
    m1, m2, m3, m4 = lm1[...], lm2[...], lm3[...], lm4[...]
    w1, w2, w3, w4 = lp1[...], lp2[...], lp3[...], lp4[...]
    s1, s2, s3, s4 = lv1[...], lv2[...], lv3[...], lv4[...]
    nl = w1 * m1 + w2 * m2 + w3 * m3 + w4 * m4
    al = w1 * s1 + w2 * s2 + w3 * s3 + w4 * s4
    ep = (w1 * (m1 - nl) ** 2 + w2 * (m2 - nl) ** 2
          + w3 * (m3 - nl) ** 2 + w4 * (m4 - nl) ** 2)

    pr = prior_ref[...]
    prw = jnp.roll(pr, -2, axis=0)
    nlw = jnp.roll(nl, -2, axis=1)
    wh = prw[None] * jnp.exp(nlw * 0.2)
    x1y1 = pr[None] + nl * 0.1 * prw[None] - 0.5 * wh
    x2y2 = jnp.roll(x1y1 + wh, 2, axis=1)
    c_idx = lax.broadcasted_iota(jnp.int32, nl.shape, 1)
    dec = jnp.where(c_idx < 2, x1y1, x2y2)

    for c in range(4):
        out_ref[c] = dec[:, c, :]
        out_ref[4 + c] = al[:, c, :]
        out_ref[8 + c] = ep[:, c, :]

    a1, a2, a3, a4 = cm1[...], cm2[...], cm3[...], cm4[...]
    q1, q2, q3, q4 = cp1[...], cp2[...], cp3[...], cp4[...]
    t1, t2, t3, t4 = cv1[...], cv2[...], cv3[...], cv4[...]
    nc = q1 * a1 + q2 * a2 + q3 * a3 + q4 * a4
    ca = q1 * t1 + q2 * t2 + q3 * t3 + q4 * t4
    ce = (q1 * (a1 - nc) ** 2 + q2 * (a2 - nc) ** 2
          + q3 * (a3 - nc) ** 2 + q4 * (a4 - nc) ** 2)
    out_ref[12:33] = nc
    out_ref[33:54] = ca
    out_ref[54:75] = ce


def kernel(prior_data, loc_mu_1, loc_var_1, loc_pi_1, loc_mu_2, loc_var_2,
           loc_pi_2, loc_mu_3, loc_var_3, loc_pi_3, loc_mu_4, loc_var_4,
           loc_pi_4, conf_mu_1, conf_var_1, conf_pi_1, conf_mu_2, conf_var_2,
           conf_pi_2, conf_mu_3, conf_var_3, conf_pi_3, conf_mu_4, conf_var_4,
           conf_pi_4):
    locs = [loc_mu_1, loc_var_1, loc_pi_1, loc_mu_2, loc_var_2, loc_pi_2,
            loc_mu_3, loc_var_3, loc_pi_3, loc_mu_4, loc_var_4, loc_pi_4]
    confs = [conf_mu_1, conf_var_1, conf_pi_1, conf_mu_2, conf_var_2,
             conf_pi_2, conf_mu_3, conf_var_3, conf_pi_3, conf_mu_4,
             conf_var_4, conf_pi_4]
    locs = [jnp.transpose(x, (0, 2, 1)) for x in locs]
    confs = [jnp.transpose(x, (2, 0, 1)) for x in confs]
    prior_t = prior_data.T

    loc_spec = pl.BlockSpec((_BN, 4, _PBL), lambda b, j: (b, 0, j))
    conf_spec = pl.BlockSpec((_C, _BN, _PBL), lambda b, j: (0, b, j))
    prior_spec = pl.BlockSpec((4, _PBL), lambda b, j: (0, j))
    out_t = pl.pallas_call(
        _body,
        grid=(_GB, _GP),
        in_specs=[prior_spec] + [loc_spec] * 12 + [conf_spec] * 12,
        out_specs=pl.BlockSpec((75, _BN, _PBL), lambda b, j: (0, b, j)),
        out_shape=jax.ShapeDtypeStruct((75, _NUM, _P), jnp.float32),
        compiler_params=pltpu.CompilerParams(
            dimension_semantics=("parallel", "arbitrary")),
    )(prior_t, *locs, *confs)
    return jnp.transpose(out_t, (1, 2, 0))


# speedup vs baseline: 1.1412x; 1.0056x over previous
import jax
import jax.numpy as jnp
from jax import lax
from jax.experimental import pallas as pl
from jax.experimental.pallas import tpu as pltpu

_NUM, _P, _C = 32, 8732, 21
_BN = 32
_PBL = 384
_GB, _GP = _NUM // _BN, 23


def _body(prior_ref,
          lm1, lv1, lp1, lm2, lv2, lp2, lm3, lv3, lp3, lm4, lv4, lp4,
          cm1, cv1, cp1, cm2, cv2, cp2, cm3, cv3, cp3, cm4, cv4, cp4,
          out_ref):
    m1, m2, m3, m4 = lm1[...], lm2[...], lm3[...], lm4[...]
    w1, w2, w3, w4 = lp1[...], lp2[...], lp3[...], lp4[...]
    s1, s2, s3, s4 = lv1[...], lv2[...], lv3[...], lv4[...]
    nl = w1 * m1 + w2 * m2 + w3 * m3 + w4 * m4
    al = w1 * s1 + w2 * s2 + w3 * s3 + w4 * s4
    ep = (w1 * (m1 - nl) ** 2 + w2 * (m2 - nl) ** 2
          + w3 * (m3 - nl) ** 2 + w4 * (m4 - nl) ** 2)

    pr = prior_ref[...]
    prw = jnp.roll(pr, -2, axis=0)
    nlw = jnp.roll(nl, -2, axis=1)
    wh = prw[None] * jnp.exp(nlw * 0.2)
    x1y1 = pr[None] + nl * 0.1 * prw[None] - 0.5 * wh
    x2y2 = jnp.roll(x1y1 + wh, 2, axis=1)
    c_idx = lax.broadcasted_iota(jnp.int32, nl.shape, 1)
    dec = jnp.where(c_idx < 2, x1y1, x2y2)

    for c in range(4):
        out_ref[c] = dec[:, c, :]
        out_ref[4 + c] = al[:, c, :]
        out_ref[8 + c] = ep[:, c, :]

    a1, a2, a3, a4 = cm1[...], cm2[...], cm3[...], cm4[...]
    q1, q2, q3, q4 = cp1[...], cp2[...], cp3[...], cp4[...]
    t1, t2, t3, t4 = cv1[...], cv2[...], cv3[...], cv4[...]
    nc = q1 * a1 + q2 * a2 + q3 * a3 + q4 * a4
    ca = q1 * t1 + q2 * t2 + q3 * t3 + q4 * t4
    ce = (q1 * (a1 - nc) ** 2 + q2 * (a2 - nc) ** 2
          + q3 * (a3 - nc) ** 2 + q4 * (a4 - nc) ** 2)
    out_ref[12:33] = nc
    out_ref[33:54] = ca
    out_ref[54:75] = ce


def kernel(prior_data, loc_mu_1, loc_var_1, loc_pi_1, loc_mu_2, loc_var_2,
           loc_pi_2, loc_mu_3, loc_var_3, loc_pi_3, loc_mu_4, loc_var_4,
           loc_pi_4, conf_mu_1, conf_var_1, conf_pi_1, conf_mu_2, conf_var_2,
           conf_pi_2, conf_mu_3, conf_var_3, conf_pi_3, conf_mu_4, conf_var_4,
           conf_pi_4):
    locs = [loc_mu_1, loc_var_1, loc_pi_1, loc_mu_2, loc_var_2, loc_pi_2,
            loc_mu_3, loc_var_3, loc_pi_3, loc_mu_4, loc_var_4, loc_pi_4]
    confs = [conf_mu_1, conf_var_1, conf_pi_1, conf_mu_2, conf_var_2,
             conf_pi_2, conf_mu_3, conf_var_3, conf_pi_3, conf_mu_4,
             conf_var_4, conf_pi_4]
    locs = [jnp.transpose(x, (0, 2, 1)) for x in locs]
    confs = [jnp.transpose(x, (2, 0, 1)) for x in confs]
    prior_t = prior_data.T

    loc_spec = pl.BlockSpec((_BN, 4, _PBL), lambda b, j: (b, 0, j))
    conf_spec = pl.BlockSpec((_C, _BN, _PBL), lambda b, j: (0, b, j))
    prior_spec = pl.BlockSpec((4, _PBL), lambda b, j: (0, j))
    out_t = pl.pallas_call(
        _body,
        grid=(_GB, _GP),
        in_specs=[prior_spec] + [loc_spec] * 12 + [conf_spec] * 12,
        out_specs=pl.BlockSpec((75, _BN, _PBL), lambda b, j: (0, b, j)),
        out_shape=jax.ShapeDtypeStruct((75, _NUM, _P), jnp.float32),
        compiler_params=pltpu.CompilerParams(
            dimension_semantics=("parallel", "arbitrary")),
    )(prior_t, *locs, *confs)
    return jnp.transpose(out_t, (1, 2, 0))
